# baseline (speedup 1.0000x reference)
import jax
import jax.numpy as jnp
from jax.experimental import pallas as pl


def _deg_body(adj_ref, deg_ref):
    deg_ref[:] = jnp.sum(adj_ref[:], axis=1, keepdims=True)


def _x1_body(adj_ref, xs_ref, d_ref, o_ref):
    o_ref[:] = -d_ref[:] * jnp.dot(adj_ref[:], xs_ref[:],
                                   preferred_element_type=jnp.float32)


def _supp_body(adj_ref, y1_ref, x0_ref, x1_ref, d_ref,
               w0_ref, w1_ref, w2_ref, bc_ref, wo_ref, o_ref):
    x2 = (-2.0 * d_ref[:] * jnp.dot(adj_ref[:], y1_ref[:],
                                    preferred_element_type=jnp.float32)
          - x0_ref[:])
    h = (jnp.dot(x0_ref[:], w0_ref[:], preferred_element_type=jnp.float32)
         + jnp.dot(x1_ref[:], w1_ref[:], preferred_element_type=jnp.float32)
         + jnp.dot(x2, w2_ref[:], preferred_element_type=jnp.float32)
         + bc_ref[:])
    h = jnp.maximum(h, 0.0)
    o_ref[:] = jnp.dot(h, wo_ref[:], preferred_element_type=jnp.float32)


def _pool_body(adj_ref, s_ref, o_ref):
    i = pl.program_id(0)
    part = jnp.dot(adj_ref[:], s_ref[:], preferred_element_type=jnp.float32)
    m = jnp.max(part, axis=0, keepdims=True)

    @pl.when(i == 0)
    def _init():
        o_ref[:] = m

    @pl.when(i != 0)
    def _acc():
        o_ref[:] = jnp.maximum(o_ref[:], m)


def kernel(x, adj, W_cheb, b_cheb, W2, b2):
    N, F = x.shape
    H = W_cheb.shape[2]
    C = W2.shape[1]
    R = next((r for r in (400, 200, 80, 40, 16, 8) if N % r == 0), N)
    G = N // R

    deg = pl.pallas_call(
        _deg_body,
        grid=(G,),
        in_specs=[pl.BlockSpec((R, N), lambda i: (i, 0))],
        out_specs=pl.BlockSpec((R, 1), lambda i: (i, 0)),
        out_shape=jax.ShapeDtypeStruct((N, 1), jnp.float32),
    )(adj)

    d = jnp.where(deg > 0, jax.lax.rsqrt(jnp.maximum(deg, 1e-12)), 0.0)
    xs = x * d

    X1 = pl.pallas_call(
        _x1_body,
        grid=(G,),
        in_specs=[
            pl.BlockSpec((R, N), lambda i: (i, 0)),
            pl.BlockSpec((N, F), lambda i: (0, 0)),
            pl.BlockSpec((R, 1), lambda i: (i, 0)),
        ],
        out_specs=pl.BlockSpec((R, F), lambda i: (i, 0)),
        out_shape=jax.ShapeDtypeStruct((N, F), jnp.float32),
    )(adj, xs, d)

    y1 = X1 * d

    support = pl.pallas_call(
        _supp_body,
        grid=(G,),
        in_specs=[
            pl.BlockSpec((R, N), lambda i: (i, 0)),
            pl.BlockSpec((N, F), lambda i: (0, 0)),
            pl.BlockSpec((R, F), lambda i: (i, 0)),
            pl.BlockSpec((R, F), lambda i: (i, 0)),
            pl.BlockSpec((R, 1), lambda i: (i, 0)),
            pl.BlockSpec((F, H), lambda i: (0, 0)),
            pl.BlockSpec((F, H), lambda i: (0, 0)),
            pl.BlockSpec((F, H), lambda i: (0, 0)),
            pl.BlockSpec((1, H), lambda i: (0, 0)),
            pl.BlockSpec((H, C), lambda i: (0, 0)),
        ],
        out_specs=pl.BlockSpec((R, C), lambda i: (i, 0)),
        out_shape=jax.ShapeDtypeStruct((N, C), jnp.float32),
    )(adj, y1, x, X1, d, W_cheb[0], W_cheb[1], W_cheb[2],
      b_cheb.reshape(1, H), W2)

    pooled = pl.pallas_call(
        _pool_body,
        grid=(G,),
        in_specs=[
            pl.BlockSpec((R, N), lambda i: (i, 0)),
            pl.BlockSpec((N, C), lambda i: (0, 0)),
        ],
        out_specs=pl.BlockSpec((1, C), lambda i: (0, 0)),
        out_shape=jax.ShapeDtypeStruct((1, C), jnp.float32),
    )(adj, support)

    return (pooled + b2)[None, :, :]


# speedup vs baseline: 1.2095x; 1.2095x over previous
import jax
import jax.numpy as jnp
from jax.experimental import pallas as pl


def _deg_body(adj_ref, deg_ref, adjb_ref):
    a = adj_ref[:]
    deg_ref[:] = jnp.sum(a, axis=1, keepdims=True)
    adjb_ref[:] = a.astype(jnp.bfloat16)


def _x1_body(adj_ref, xs_ref, d_ref, o_ref):
    o_ref[:] = -d_ref[:] * jnp.dot(adj_ref[:], xs_ref[:],
                                   preferred_element_type=jnp.float32)


def _supp_body(adj_ref, y1_ref, x0_ref, x1_ref, d_ref,
               w0_ref, w1_ref, w2_ref, bc_ref, wo_ref, o_ref):
    x2 = (-2.0 * d_ref[:] * jnp.dot(adj_ref[:], y1_ref[:],
                                    preferred_element_type=jnp.float32)
          - x0_ref[:])
    h = (jnp.dot(x0_ref[:], w0_ref[:], preferred_element_type=jnp.float32)
         + jnp.dot(x1_ref[:], w1_ref[:], preferred_element_type=jnp.float32)
         + jnp.dot(x2, w2_ref[:], preferred_element_type=jnp.float32)
         + bc_ref[:])
    h = jnp.maximum(h, 0.0)
    o_ref[:] = jnp.dot(h, wo_ref[:], preferred_element_type=jnp.float32)


def _pool_body(adj_ref, s_ref, o_ref):
    i = pl.program_id(0)
    part = jnp.dot(adj_ref[:], s_ref[:], preferred_element_type=jnp.float32)
    m = jnp.max(part, axis=0, keepdims=True)

    @pl.when(i == 0)
    def _init():
        o_ref[:] = m

    @pl.when(i != 0)
    def _acc():
        o_ref[:] = jnp.maximum(o_ref[:], m)


def kernel(x, adj, W_cheb, b_cheb, W2, b2):
    N, F = x.shape
    H = W_cheb.shape[2]
    C = W2.shape[1]
    R = next((r for r in (400, 200, 80, 40, 16, 8) if N % r == 0), N)
    G = N // R

    deg, adjb = pl.pallas_call(
        _deg_body,
        grid=(G,),
        in_specs=[pl.BlockSpec((R, N), lambda i: (i, 0))],
        out_specs=[pl.BlockSpec((R, 1), lambda i: (i, 0)),
                   pl.BlockSpec((R, N), lambda i: (i, 0))],
        out_shape=[jax.ShapeDtypeStruct((N, 1), jnp.float32),
                   jax.ShapeDtypeStruct((N, N), jnp.bfloat16)],
    )(adj)

    d = jnp.where(deg > 0, jax.lax.rsqrt(jnp.maximum(deg, 1e-12)), 0.0)
    xs = (x * d).astype(jnp.bfloat16)

    X1 = pl.pallas_call(
        _x1_body,
        grid=(G,),
        in_specs=[
            pl.BlockSpec((R, N), lambda i: (i, 0)),
            pl.BlockSpec((N, F), lambda i: (0, 0)),
            pl.BlockSpec((R, 1), lambda i: (i, 0)),
        ],
        out_specs=pl.BlockSpec((R, F), lambda i: (i, 0)),
        out_shape=jax.ShapeDtypeStruct((N, F), jnp.float32),
    )(adjb, xs, d)

    y1 = (X1 * d).astype(jnp.bfloat16)

    support = pl.pallas_call(
        _supp_body,
        grid=(G,),
        in_specs=[
            pl.BlockSpec((R, N), lambda i: (i, 0)),
            pl.BlockSpec((N, F), lambda i: (0, 0)),
            pl.BlockSpec((R, F), lambda i: (i, 0)),
            pl.BlockSpec((R, F), lambda i: (i, 0)),
            pl.BlockSpec((R, 1), lambda i: (i, 0)),
            pl.BlockSpec((F, H), lambda i: (0, 0)),
            pl.BlockSpec((F, H), lambda i: (0, 0)),
            pl.BlockSpec((F, H), lambda i: (0, 0)),
            pl.BlockSpec((1, H), lambda i: (0, 0)),
            pl.BlockSpec((H, C), lambda i: (0, 0)),
        ],
        out_specs=pl.BlockSpec((R, C), lambda i: (i, 0)),
        out_shape=jax.ShapeDtypeStruct((N, C), jnp.float32),
    )(adjb, y1, x, X1, d, W_cheb[0], W_cheb[1], W_cheb[2],
      b_cheb.reshape(1, H), W2)

    pooled = pl.pallas_call(
        _pool_body,
        grid=(G,),
        in_specs=[
            pl.BlockSpec((R, N), lambda i: (i, 0)),
            pl.BlockSpec((N, C), lambda i: (0, 0)),
        ],
        out_specs=pl.BlockSpec((1, C), lambda i: (0, 0)),
        out_shape=jax.ShapeDtypeStruct((1, C), jnp.float32),
    )(adjb, support.astype(jnp.bfloat16))

    return (pooled + b2)[None, :, :]
